# baseline (device time: 44176 ns/iter reference)
import jax
import jax.numpy as jnp
from jax import lax
from jax.experimental import pallas as pl
from jax.experimental.pallas import tpu as pltpu

N_DEV = 4
B, SQ, HQ, DH = 2, 512, 8, 64
SKV_LOC = 512
D_MODEL = 768
BH = B * HQ
BLK = 64
CHUNK = SQ // N_DEV
NPAIR = BH // 2
HPAIR = HQ // 2
NSLAB = NPAIR + 1


def kernel(x, Wq, K_ext, V_ext, Wo):
    bf16 = jnp.bfloat16
    K2 = K_ext.transpose(0, 2, 1, 3).reshape(BH, SKV_LOC, DH).astype(bf16)
    V2 = V_ext.transpose(0, 2, 1, 3).reshape(BH, SKV_LOC, DH).astype(bf16)
    V3 = jnp.concatenate(
        [V2, jnp.ones((BH, SKV_LOC, 1), bf16),
         jnp.zeros((BH, SKV_LOC, DH - 1), bf16)], axis=2)
    Wq2 = (Wq * 0.125).reshape(D_MODEL, HPAIR, 2 * DH).transpose(1, 0, 2)
    Wq2 = Wq2.astype(bf16)
    x16 = x.astype(bf16)
    Wo16 = Wo.astype(bf16)

    def body(x_ref, wq_ref, k_ref, v_ref, wo_ref, out_ref,
             rs_o, p_o, ctx_s, ag,
             rso_send, rso_recv, ag_send, ag_recv):
        my = lax.axis_index("i")
        left = lax.rem(my + N_DEV - 1, N_DEV)
        right = lax.rem(my + 1, N_DEV)

        barrier = pltpu.get_barrier_semaphore()
        for nbr in (left, right):
            pl.semaphore_signal(barrier, inc=1, device_id=(nbr,),
                                device_id_type=pl.DeviceIdType.MESH)
        pl.semaphore_wait(barrier, 2)

        def compute_partial(c, o_dst, o_slot, o_dtype):
            qb = (lax.broadcasted_iota(jnp.int32, (CHUNK, SKV_LOC), 0)
                  + c * CHUNK) // BLK
            kb = (lax.broadcasted_iota(jnp.int32, (CHUNK, SKV_LOC), 1) // BLK
                  + my * (SKV_LOC // BLK))
            m = (qb == kb) | (kb == 0) | (((qb + kb) % 3) == 0)
            bias = jnp.where(m, 0.0, -1e30)
            for b in range(B):
                xb = x_ref[b, pl.ds(c * CHUNK, CHUNK), :]
                for hp in range(HPAIR):
                    q2 = jnp.dot(xb, wq_ref[hp],
                                 preferred_element_type=jnp.float32
                                 ).astype(bf16)
                    for sub in range(2):
                        h = 2 * hp + sub
                        bh = b * HQ + h
                        pair, off = bh // 2, sub * DH
                        q = q2[:, off:off + DH]
                        s = lax.dot_general(
                            q, k_ref[bh], (((1,), (1,)), ((), ())),
                            preferred_element_type=jnp.float32,
                        )
                        w = jnp.exp(s + bias).astype(bf16)
                        o_ext = jnp.dot(w, v_ref[bh],
                                        preferred_element_type=jnp.float32)
                        o_dst[o_slot, pair, :, off:off + DH] = (
                            o_ext[:, :DH].astype(o_dtype))
                        o_dst[o_slot, NPAIR, :, bh:bh + 1] = (
                            o_ext[:, DH:DH + 1].astype(o_dtype))

        compute_partial(lax.rem(my + N_DEV - 1, N_DEV), rs_o, 0, bf16)
        for t in range(N_DEV - 1):
            ss, rs = t % 2, (t + 1) % 2
            ro = pltpu.make_async_remote_copy(
                src_ref=rs_o.at[ss], dst_ref=rs_o.at[rs],
                send_sem=rso_send.at[ss], recv_sem=rso_recv.at[rs],
                device_id=(right,), device_id_type=pl.DeviceIdType.MESH)
            ro.start()
            compute_partial(lax.rem(my + 2 * N_DEV - 2 - t, N_DEV),
                            p_o, 0, jnp.float32)
            ro.wait()
            rs_o[rs] = (rs_o[rs].astype(jnp.float32)
                        + p_o[0]).astype(bf16)
        fin = (N_DEV - 1) % 2

        r1 = []
        for b in range(B):
            for h in range(HQ):
                bh = b * HQ + h
                pair, off = bh // 2, (bh % 2) * DH
                lcol = rs_o[fin, NPAIR, :, bh:bh + 1].astype(jnp.float32)
                ctx_s[b, :, h * DH:(h + 1) * DH] = (
                    rs_o[fin, pair, :, off:off + DH].astype(jnp.float32)
                    / lcol).astype(bf16)
            oc = jnp.dot(ctx_s[b], wo_ref[:, :],
                         preferred_element_type=jnp.float32)
            ag[0, b] = oc.astype(bf16)
            out_ref[b, pl.ds(my * CHUNK, CHUNK), :] = oc
            a_r = pltpu.make_async_remote_copy(
                src_ref=ag.at[0, b], dst_ref=ag.at[1, b],
                send_sem=ag_send.at[2 * b], recv_sem=ag_recv.at[2 * b],
                device_id=(right,), device_id_type=pl.DeviceIdType.MESH)
            a_l = pltpu.make_async_remote_copy(
                src_ref=ag.at[0, b], dst_ref=ag.at[2, b],
                send_sem=ag_send.at[2 * b + 1], recv_sem=ag_recv.at[2 * b + 1],
                device_id=(left,), device_id_type=pl.DeviceIdType.MESH)
            a_r.start()
            a_l.start()
            r1.append((a_r, a_l))
        r1[0][0].wait()
        fwd_r = pltpu.make_async_remote_copy(
            src_ref=ag.at[1, 0], dst_ref=ag.at[3, 0],
            send_sem=ag_send.at[4], recv_sem=ag_recv.at[4],
            device_id=(right,), device_id_type=pl.DeviceIdType.MESH)
        fwd_r.start()
        r1[1][1].wait()
        fwd_l = pltpu.make_async_remote_copy(
            src_ref=ag.at[2, 1], dst_ref=ag.at[3, 1],
            send_sem=ag_send.at[5], recv_sem=ag_recv.at[5],
            device_id=(left,), device_id_type=pl.DeviceIdType.MESH)
        fwd_l.start()
        r1[0][1].wait()
        r1[1][0].wait()
        for slot, origin in ((1, left), (2, right)):
            for b in range(B):
                out_ref[b, pl.ds(origin * CHUNK, CHUNK), :] = (
                    ag[slot, b].astype(jnp.float32))
        fwd_r.wait()
        fwd_l.wait()
        opp = lax.rem(my + 2, N_DEV)
        for b in range(B):
            out_ref[b, pl.ds(opp * CHUNK, CHUNK), :] = (
                ag[3, b].astype(jnp.float32))

    return pl.pallas_call(
        body,
        out_shape=jax.ShapeDtypeStruct((B, SQ, D_MODEL), jnp.float32),
        in_specs=[pl.BlockSpec(memory_space=pltpu.VMEM)] * 5,
        out_specs=pl.BlockSpec(memory_space=pltpu.VMEM),
        scratch_shapes=[
            pltpu.VMEM((2, NSLAB, CHUNK, 2 * DH), jnp.bfloat16),
            pltpu.VMEM((1, NSLAB, CHUNK, 2 * DH), jnp.float32),
            pltpu.VMEM((B, CHUNK, HQ * DH), jnp.bfloat16),
            pltpu.VMEM((4, B, CHUNK, D_MODEL), jnp.bfloat16),
            pltpu.SemaphoreType.DMA((2,)),
            pltpu.SemaphoreType.DMA((2,)),
            pltpu.SemaphoreType.DMA((6,)),
            pltpu.SemaphoreType.DMA((6,)),
        ],
        compiler_params=pltpu.CompilerParams(collective_id=0),
    )(x16, Wq2, K2, V3, Wo16)


# device time: 34807 ns/iter; 1.2692x vs baseline; 1.2692x over previous
import os

import jax
import jax.numpy as jnp
from jax import lax
from jax.experimental import pallas as pl
from jax.experimental.pallas import tpu as pltpu

PROBE = int(os.environ.get("PROBE", "0"))

N_DEV = 4
B, SQ, HQ, DH = 2, 512, 8, 64
SKV_LOC = 512
D_MODEL = 768
BH = B * HQ
BLK = 64
CHUNK = SQ // N_DEV
NPAIR = BH // 2
HPAIR = HQ // 2
NSLAB = NPAIR + 1


def kernel(x, Wq, K_ext, V_ext, Wo):
    bf16 = jnp.bfloat16
    K2 = K_ext.transpose(0, 2, 1, 3).reshape(BH, SKV_LOC, DH).astype(bf16)
    V2 = V_ext.transpose(0, 2, 1, 3).reshape(BH, SKV_LOC, DH).astype(bf16)
    V3 = jnp.concatenate(
        [V2, jnp.ones((BH, SKV_LOC, 1), bf16),
         jnp.zeros((BH, SKV_LOC, DH - 1), bf16)], axis=2)
    Wq2 = (Wq * 0.125).reshape(D_MODEL, HPAIR, 2 * DH).transpose(1, 0, 2)
    Wq2 = Wq2.astype(bf16)
    x16 = x.astype(bf16)
    Wo16 = Wo.astype(bf16)

    def body(x_ref, wq_ref, k_ref, v_ref, wo_ref, out_ref,
             rs_o, p_o, ctx_s, ag,
             rso_send, rso_recv, ag_send, ag_recv):
        my = lax.axis_index("i")
        left = lax.rem(my + N_DEV - 1, N_DEV)
        right = lax.rem(my + 1, N_DEV)

        barrier = pltpu.get_barrier_semaphore()
        for nbr in (left, right):
            pl.semaphore_signal(barrier, inc=1, device_id=(nbr,),
                                device_id_type=pl.DeviceIdType.MESH)
        pl.semaphore_wait(barrier, 2)

        def compute_partial(c, o_dst, o_slot, o_dtype):
            qb = (lax.broadcasted_iota(jnp.int32, (CHUNK, SKV_LOC), 0)
                  + c * CHUNK) // BLK
            kb = (lax.broadcasted_iota(jnp.int32, (CHUNK, SKV_LOC), 1) // BLK
                  + my * (SKV_LOC // BLK))
            m = (qb == kb) | (kb == 0) | (((qb + kb) % 3) == 0)
            bias = jnp.where(m, 0.0, -1e30)
            for b in range(B):
                xb = x_ref[b, pl.ds(c * CHUNK, CHUNK), :]
                for hp in range(HPAIR):
                    q2 = jnp.dot(xb, wq_ref[hp],
                                 preferred_element_type=jnp.float32
                                 ).astype(bf16)
                    for sub in range(2):
                        h = 2 * hp + sub
                        bh = b * HQ + h
                        pair, off = bh // 2, sub * DH
                        q = q2[:, off:off + DH]
                        s = lax.dot_general(
                            q, k_ref[bh], (((1,), (1,)), ((), ())),
                            preferred_element_type=jnp.float32,
                        )
                        w = jnp.exp(s + bias).astype(bf16)
                        o_ext = jnp.dot(w, v_ref[bh],
                                        preferred_element_type=jnp.float32)
                        o_dst[o_slot, pair, :, off:off + DH] = (
                            o_ext[:, :DH].astype(o_dtype))
                        o_dst[o_slot, NPAIR, :, bh:bh + 1] = (
                            o_ext[:, DH:DH + 1].astype(o_dtype))

        compute_partial(lax.rem(my + N_DEV - 1, N_DEV), rs_o, 0, bf16)
        for t in range(0 if PROBE == 1 else N_DEV - 1):
            ss, rs = t % 2, (t + 1) % 2
            ro = pltpu.make_async_remote_copy(
                src_ref=rs_o.at[ss], dst_ref=rs_o.at[rs],
                send_sem=rso_send.at[ss], recv_sem=rso_recv.at[rs],
                device_id=(right,), device_id_type=pl.DeviceIdType.MESH)
            ro.start()
            compute_partial(lax.rem(my + 2 * N_DEV - 2 - t, N_DEV),
                            p_o, 0, jnp.float32)
            ro.wait()
            rs_o[rs] = (rs_o[rs].astype(jnp.float32)
                        + p_o[0]).astype(bf16)
        fin = (N_DEV - 1) % 2

        r1 = []
        for b in range(B):
            for h in range(HQ):
                bh = b * HQ + h
                pair, off = bh // 2, (bh % 2) * DH
                lcol = rs_o[fin, NPAIR, :, bh:bh + 1].astype(jnp.float32)
                ctx_s[b, :, h * DH:(h + 1) * DH] = (
                    rs_o[fin, pair, :, off:off + DH].astype(jnp.float32)
                    / lcol).astype(bf16)
            oc = jnp.dot(ctx_s[b], wo_ref[:, :],
                         preferred_element_type=jnp.float32)
            ag[0, b] = oc.astype(bf16)
            out_ref[b, pl.ds(my * CHUNK, CHUNK), :] = oc
            if PROBE:
                continue
            a_r = pltpu.make_async_remote_copy(
                src_ref=ag.at[0, b], dst_ref=ag.at[1, b],
                send_sem=ag_send.at[2 * b], recv_sem=ag_recv.at[2 * b],
                device_id=(right,), device_id_type=pl.DeviceIdType.MESH)
            a_l = pltpu.make_async_remote_copy(
                src_ref=ag.at[0, b], dst_ref=ag.at[2, b],
                send_sem=ag_send.at[2 * b + 1], recv_sem=ag_recv.at[2 * b + 1],
                device_id=(left,), device_id_type=pl.DeviceIdType.MESH)
            a_r.start()
            a_l.start()
            r1.append((a_r, a_l))
        if not PROBE:
            r1[0][0].wait()
            fwd_r = pltpu.make_async_remote_copy(
                src_ref=ag.at[1, 0], dst_ref=ag.at[3, 0],
                send_sem=ag_send.at[4], recv_sem=ag_recv.at[4],
                device_id=(right,), device_id_type=pl.DeviceIdType.MESH)
            fwd_r.start()
            r1[1][1].wait()
            fwd_l = pltpu.make_async_remote_copy(
                src_ref=ag.at[2, 1], dst_ref=ag.at[3, 1],
                send_sem=ag_send.at[5], recv_sem=ag_recv.at[5],
                device_id=(left,), device_id_type=pl.DeviceIdType.MESH)
            fwd_l.start()
            r1[0][1].wait()
            r1[1][0].wait()
        for slot, origin in ((1, left), (2, right)):
            for b in range(B):
                out_ref[b, pl.ds(origin * CHUNK, CHUNK), :] = (
                    ag[slot, b].astype(jnp.float32))
        if not PROBE:
            fwd_r.wait()
            fwd_l.wait()
        opp = lax.rem(my + 2, N_DEV)
        for b in range(B):
            out_ref[b, pl.ds(opp * CHUNK, CHUNK), :] = (
                ag[3, b].astype(jnp.float32))

    return pl.pallas_call(
        body,
        out_shape=jax.ShapeDtypeStruct((B, SQ, D_MODEL), jnp.float32),
        in_specs=[pl.BlockSpec(memory_space=pltpu.VMEM)] * 5,
        out_specs=pl.BlockSpec(memory_space=pltpu.VMEM),
        scratch_shapes=[
            pltpu.VMEM((2, NSLAB, CHUNK, 2 * DH), jnp.bfloat16),
            pltpu.VMEM((1, NSLAB, CHUNK, 2 * DH), jnp.float32),
            pltpu.VMEM((B, CHUNK, HQ * DH), jnp.bfloat16),
            pltpu.VMEM((4, B, CHUNK, D_MODEL), jnp.bfloat16),
            pltpu.SemaphoreType.DMA((2,)),
            pltpu.SemaphoreType.DMA((2,)),
            pltpu.SemaphoreType.DMA((6,)),
            pltpu.SemaphoreType.DMA((6,)),
        ],
        compiler_params=pltpu.CompilerParams(collective_id=0),
    )(x16, Wq2, K2, V3, Wo16)
